# baseline (speedup 1.0000x reference)
import functools

import jax
import jax.numpy as jnp
from jax import lax
from jax.experimental import pallas as pl
from jax.experimental.pallas import tpu as pltpu
from jax.experimental.pallas import tpu_sc as plsc

NC, NS = 2, 16
NW = NC * NS
SPLITS = ((0, 104), (104, 96))
BLK = 4


@functools.partial(jax.jit, static_argnames=("b", "s", "d"))
def _sc_gather(x, table, b, s, d):
    rows_per_w = b // NW
    n_blocks = rows_per_w // 2
    cmax = max(n for _, n in SPLITS)
    mesh = plsc.VectorSubcoreMesh(core_axis_name="c", subcore_axis_name="s")

    @functools.partial(
        pl.kernel,
        mesh=mesh,
        out_type=jax.ShapeDtypeStruct((b, s, d), jnp.float32),
        scratch_types=[
            pltpu.VMEM((rows_per_w, s), jnp.int32),
            pltpu.VMEM((2 * BLK, cmax, d), jnp.float32),
            pltpu.SemaphoreType.DMA,
            pltpu.SemaphoreType.DMA,
        ],
        compiler_params=pltpu.CompilerParams(use_tc_tiling_on_sc=False),
    )
    def k(idx_hbm, table_hbm, out_hbm, idx_v, rows_v, gsem, osem):
        wid = lax.axis_index("s") * NC + lax.axis_index("c")
        row0 = wid * rows_per_w
        pltpu.sync_copy(idx_hbm.at[pl.ds(row0, rows_per_w)], idx_v)

        def gather(r2, i, slot):
            off, n = SPLITS[i % 2]
            pltpu.async_copy(
                table_hbm.at[idx_v.at[r2 + i // 2, pl.ds(off, n)]],
                rows_v.at[slot, pl.ds(0, n)],
                gsem,
            )

        def put(r2, i, slot):
            off, n = SPLITS[i % 2]
            pltpu.async_copy(
                rows_v.at[slot, pl.ds(0, n)],
                out_hbm.at[row0 + r2 + i // 2, pl.ds(off, n)],
                osem,
            )

        def drain(sem):
            for i in range(BLK):
                _, n = SPLITS[i % 2]
                pltpu.make_async_copy(
                    rows_v.at[0, pl.ds(0, n)], out_hbm.at[0, pl.ds(0, n)], sem
                ).wait()

        for i in range(BLK):
            gather(0, i, i)

        def body(g, carry):
            cur = (g % 2) * BLK

            @pl.when(g > 0)
            def _():
                drain(osem)

            drain(gsem)

            for i in range(BLK):
                put(2 * g, i, cur + i)

            @pl.when(g + 1 < n_blocks)
            def _():
                for i in range(BLK):
                    gather(2 * (g + 1), i, (BLK - cur) + i)

            return carry

        lax.fori_loop(0, n_blocks, body, None)
        drain(osem)

    return k(x, table)


def kernel(X, table):
    b, s = X.shape
    d = table.shape[1]
    return _sc_gather(X.astype(jnp.int32), table, b, s, d)
